# baseline (speedup 1.0000x reference)
import jax
import jax.numpy as jnp
from jax import lax
from jax.experimental import pallas as pl
from jax.experimental.pallas import tpu as pltpu
from jax.experimental.pallas import tpu_sc as plsc

_B, _L = 16384, 20
_DLOC, _DTIME = 64, 32
_DOUT = _DLOC + _DTIME
_NC, _NS = 2, 16
_NW = _NC * _NS
_C = 128
_B_PER_W = _B // _NW
_NBC = _B_PER_W // _C
_BP = _C + 1


def _body(xV, tT, loc0, loc1, loc2, tw0, tw1, out_hbm,
          idx_vx, idx_vt, b0, b1, b2, tb0, tb1, outT0, outT1,
          sg0, sg1, ss0, ss1):
    wid = lax.axis_index("s") * _NC + lax.axis_index("c")
    sg = (sg0, sg1)
    ss = (ss0, ss1)
    outTs = (outT0, outT1)

    def gather_descs(slot, lc):
        s = sg[slot]
        lq, lr = lc // 8, lc % 8
        return (
            pltpu.make_async_copy(loc0.at[idx_vx.at[0, lq, lr]], b0.at[slot], s),
            pltpu.make_async_copy(loc1.at[idx_vx.at[1, lq, lr]], b1.at[slot], s),
            pltpu.make_async_copy(loc2.at[idx_vx.at[2, lq, lr]], b2.at[slot], s),
            pltpu.make_async_copy(tw0.at[idx_vt.at[0, lc]], tb0.at[slot], s),
            pltpu.make_async_copy(tw1.at[idx_vt.at[1, lc]], tb1.at[slot], s),
        )

    def fire(slot, lc):
        for d in gather_descs(slot, lc):
            d.start()

    def wait_gathers(slot, lc):
        for d in gather_descs(slot, lc):
            d.wait()

    def scatter_desc(slot, lc, bblk):
        return pltpu.make_async_copy(
            outTs[slot].at[:, :, pl.ds(0, _C)],
            out_hbm.at[lc, :, bblk], ss[slot])

    lane = lax.iota(jnp.int32, 16)
    hi8 = lane // 8
    lo8 = lane % 8

    def compute(slot):
        oT = outTs[slot]

        def row(i, _):
            bi = jnp.full((16,), 0, jnp.int32) + i
            for j in range(_DLOC // 16):
                s = pl.ds(j * 16, 16)
                v = b0[slot, i, s] + b1[slot, i, s] + b2[slot, i, s]
                plsc.store_scatter(oT, [hi8 + 2 * j, lo8, bi], v)
            for j in range(_DTIME // 16):
                s = pl.ds(j * 16, 16)
                v = tb0[slot, i, s] + tb1[slot, i, s]
                plsc.store_scatter(oT, [hi8 + (8 + 2 * j), lo8, bi], v)
            return ()
        lax.fori_loop(0, _C, row, ())

    @pl.loop(0, _NBC)
    def bchunk(bc):
        b0c = wid * _B_PER_W + bc * _C
        bblk = b0c // _C
        for k in range(3):
            pltpu.sync_copy(xV.at[k, :, bblk], idx_vx.at[k])
        for k in range(2):
            pltpu.sync_copy(tT.at[:, k, pl.ds(b0c, _C)], idx_vt.at[k])
        fire(0, 0)

        @pl.loop(0, _L, step=2)
        def pair(k):
            for b in range(2):
                lc = k + b
                nxt = lc + 1

                @pl.when(nxt < _L)
                def _():
                    fire(1 - b, nxt)

                wait_gathers(b, lc)
                compute(b)

                @pl.when(bc * _L + lc >= 2)
                def _():
                    scatter_desc(b, lc, bblk).wait()

                scatter_desc(b, lc, bblk).start()

    scatter_desc(0, 0, 0).wait()
    scatter_desc(1, 0, 0).wait()


def kernel(x, t, loc_w0, loc_w1, loc_w2, time_w0, time_w1):
    xp = jnp.pad(x.astype(jnp.int32), ((0, 0), (0, 4), (0, 0)))
    xV = jnp.transpose(
        jnp.transpose(xp, (2, 1, 0)).reshape(3, 3, 8, _B // _C, _C),
        (0, 1, 3, 2, 4))
    tT = jnp.transpose(t, (1, 2, 0)).astype(jnp.int32)
    mesh = plsc.VectorSubcoreMesh(core_axis_name="c", subcore_axis_name="s",
                                  num_cores=_NC, num_subcores=_NS)
    out5 = pl.kernel(
        _body,
        out_type=jax.ShapeDtypeStruct((_L, _DOUT // 8, _B // _C, 8, _C),
                                      jnp.float32),
        mesh=mesh,
        scratch_types=[
            pltpu.VMEM((3, 3, 8, _C), jnp.int32),
            pltpu.VMEM((2, _L, _C), jnp.int32),
            pltpu.VMEM((2, _C, _DLOC), jnp.float32),
            pltpu.VMEM((2, _C, _DLOC), jnp.float32),
            pltpu.VMEM((2, _C, _DLOC), jnp.float32),
            pltpu.VMEM((2, _C, _DTIME), jnp.float32),
            pltpu.VMEM((2, _C, _DTIME), jnp.float32),
            pltpu.VMEM((_DOUT // 8, 8, _BP), jnp.float32),
            pltpu.VMEM((_DOUT // 8, 8, _BP), jnp.float32),
            pltpu.SemaphoreType.DMA,
            pltpu.SemaphoreType.DMA,
            pltpu.SemaphoreType.DMA,
            pltpu.SemaphoreType.DMA,
        ],
        compiler_params=pltpu.CompilerParams(use_tc_tiling_on_sc=False,
                                             needs_layout_passes=False),
    )(xV, tT, loc_w0, loc_w1, loc_w2, time_w0, time_w1)
    return jnp.transpose(out5, (2, 4, 0, 1, 3)).reshape(_B, _L, _DOUT)


# speedup vs baseline: 1.0148x; 1.0148x over previous
import jax
import jax.numpy as jnp
from jax import lax
from jax.experimental import pallas as pl
from jax.experimental.pallas import tpu as pltpu
from jax.experimental.pallas import tpu_sc as plsc

_B, _L = 16384, 20
_DLOC, _DTIME = 64, 32
_DOUT = _DLOC + _DTIME
_NC, _NS = 2, 16
_NW = _NC * _NS
_C = 128
_B_PER_W = _B // _NW
_NBC = _B_PER_W // _C
_BP = _C + 1


def _body(xV, tV, loc0, loc1, loc2, tw0, tw1, out_hbm,
          idx_vx, idx_vt, b0, b1, b2, tb0, tb1, outT0, outT1,
          sg0, sg1, ss0, ss1):
    wid = lax.axis_index("s") * _NC + lax.axis_index("c")
    sg = (sg0, sg1)
    ss = (ss0, ss1)
    outTs = (outT0, outT1)

    def gather_descs(slot, lc):
        s = sg[slot]
        lq, lr = lc // 8, lc % 8
        return (
            pltpu.make_async_copy(loc0.at[idx_vx.at[0, lq, lr]], b0.at[slot], s),
            pltpu.make_async_copy(loc1.at[idx_vx.at[1, lq, lr]], b1.at[slot], s),
            pltpu.make_async_copy(loc2.at[idx_vx.at[2, lq, lr]], b2.at[slot], s),
            pltpu.make_async_copy(tw0.at[idx_vt.at[lc, 0]], tb0.at[slot], s),
            pltpu.make_async_copy(tw1.at[idx_vt.at[lc, 1]], tb1.at[slot], s),
        )

    def fire(slot, lc):
        for d in gather_descs(slot, lc):
            d.start()

    def wait_gathers(slot, lc):
        for d in gather_descs(slot, lc):
            d.wait()

    def scatter_desc(slot, lc, bblk):
        return pltpu.make_async_copy(
            outTs[slot].at[:, :, pl.ds(0, _C)],
            out_hbm.at[lc, :, bblk], ss[slot])

    lane = lax.iota(jnp.int32, 16)
    hi8 = lane // 8
    lo8 = lane % 8

    def compute(slot):
        oT = outTs[slot]

        def row(i, _):
            bi = jnp.full((16,), 0, jnp.int32) + i
            for j in range(_DLOC // 16):
                s = pl.ds(j * 16, 16)
                v = b0[slot, i, s] + b1[slot, i, s] + b2[slot, i, s]
                plsc.store_scatter(oT, [hi8 + 2 * j, lo8, bi], v)
            for j in range(_DTIME // 16):
                s = pl.ds(j * 16, 16)
                v = tb0[slot, i, s] + tb1[slot, i, s]
                plsc.store_scatter(oT, [hi8 + (8 + 2 * j), lo8, bi], v)
            return ()
        lax.fori_loop(0, _C, row, ())

    @pl.loop(0, _NBC)
    def bchunk(bc):
        b0c = wid * _B_PER_W + bc * _C
        bblk = b0c // _C
        pltpu.sync_copy(xV.at[:, :, bblk], idx_vx)
        pltpu.sync_copy(tV.at[:, bblk], idx_vt)
        fire(0, 0)

        @pl.loop(0, _L, step=2)
        def pair(k):
            for b in range(2):
                lc = k + b
                nxt = lc + 1

                @pl.when(nxt < _L)
                def _():
                    fire(1 - b, nxt)

                wait_gathers(b, lc)
                compute(b)

                @pl.when(bc * _L + lc >= 2)
                def _():
                    scatter_desc(b, lc, bblk).wait()

                scatter_desc(b, lc, bblk).start()

    scatter_desc(0, 0, 0).wait()
    scatter_desc(1, 0, 0).wait()


def kernel(x, t, loc_w0, loc_w1, loc_w2, time_w0, time_w1):
    xp = jnp.pad(x.astype(jnp.int32), ((0, 0), (0, 4), (0, 0)))
    xV = jnp.transpose(
        jnp.transpose(xp, (2, 1, 0)).reshape(3, 3, 8, _B // _C, _C),
        (0, 1, 3, 2, 4))
    tV = jnp.transpose(
        jnp.transpose(t.astype(jnp.int32), (1, 0, 2)).reshape(
            _L, _B // _C, _C, 2),
        (0, 1, 3, 2))
    mesh = plsc.VectorSubcoreMesh(core_axis_name="c", subcore_axis_name="s",
                                  num_cores=_NC, num_subcores=_NS)
    out5 = pl.kernel(
        _body,
        out_type=jax.ShapeDtypeStruct((_L, _DOUT // 8, _B // _C, 8, _C),
                                      jnp.float32),
        mesh=mesh,
        scratch_types=[
            pltpu.VMEM((3, 3, 8, _C), jnp.int32),
            pltpu.VMEM((_L, 2, _C), jnp.int32),
            pltpu.VMEM((2, _C, _DLOC), jnp.float32),
            pltpu.VMEM((2, _C, _DLOC), jnp.float32),
            pltpu.VMEM((2, _C, _DLOC), jnp.float32),
            pltpu.VMEM((2, _C, _DTIME), jnp.float32),
            pltpu.VMEM((2, _C, _DTIME), jnp.float32),
            pltpu.VMEM((_DOUT // 8, 8, _BP), jnp.float32),
            pltpu.VMEM((_DOUT // 8, 8, _BP), jnp.float32),
            pltpu.SemaphoreType.DMA,
            pltpu.SemaphoreType.DMA,
            pltpu.SemaphoreType.DMA,
            pltpu.SemaphoreType.DMA,
        ],
        compiler_params=pltpu.CompilerParams(use_tc_tiling_on_sc=False,
                                             needs_layout_passes=False),
    )(xV, tV, loc_w0, loc_w1, loc_w2, time_w0, time_w1)
    return jnp.transpose(out5, (2, 4, 0, 1, 3)).reshape(_B, _L, _DOUT)
